# baseline (speedup 1.0000x reference)
import jax
import jax.numpy as jnp
from jax import lax
from jax.experimental import pallas as pl
from jax.experimental.pallas import tpu as pltpu
from jax.experimental.pallas import tpu_sc as plsc

EPS = 1e-6

N1, N2 = 8192, 2048
CO = 64
FD = 3 * CO

_NC, _NS, _L = 2, 16, 16
_NW = _NC * _NS
_QPW = N1 // _NW
_CHUNK = 64
_NCHUNK = _QPW // _CHUNK


def _vn_math(x_ref, wf_ref, wd_ref, g_ref, b_ref):
    xv = [x_ref[v] for v in range(3)]
    pv = [jnp.dot(wf_ref[...], x, preferred_element_type=jnp.float32)
          for x in xv]
    dv = [jnp.dot(wd_ref[...], x, preferred_element_type=jnp.float32)
          for x in xv]
    nrm = jnp.sqrt(pv[0] * pv[0] + pv[1] * pv[1] + pv[2] * pv[2]) + EPS
    mean = jnp.mean(nrm, axis=1, keepdims=True)
    cen = nrm - mean
    var = jnp.mean(cen * cen, axis=1, keepdims=True)
    g = jnp.reshape(g_ref[...], (CO, 1))
    b = jnp.reshape(b_ref[...], (CO, 1))
    nbn = cen / jnp.sqrt(var + 1e-5) * g + b
    scale = nbn / nrm
    dotp = (pv[0] * dv[0] + pv[1] * dv[1] + pv[2] * dv[2]) * scale
    dnsq = dv[0] * dv[0] + dv[1] * dv[1] + dv[2] * dv[2]
    coef = jnp.where(dotp < 0, 0.8 * dotp / (dnsq + EPS), 0.0)
    return [pv[v] * scale - coef * dv[v] for v in range(3)]


def _vn1_body(x_ref, wf_ref, wd_ref, g_ref, b_ref, y_ref):
    yv = _vn_math(x_ref, wf_ref, wd_ref, g_ref, b_ref)
    for v in range(3):
        y_ref[v] = yv[v]


def _vn1(xt, wf, wd, g, b):
    return pl.pallas_call(
        _vn1_body,
        out_shape=jax.ShapeDtypeStruct((3, CO, N1), jnp.float32),
    )(xt, wf, wd, g, b)


def _vn2_body(x_ref, wf_ref, wd_ref, g_ref, b_ref, ta_ref, tb_ref):
    yv = _vn_math(x_ref, wf_ref, wd_ref, g_ref, b_ref)
    ta_ref[:, 0:CO] = jnp.transpose(yv[0], (1, 0))
    ta_ref[:, CO:2 * CO] = jnp.transpose(yv[1], (1, 0))
    tb_ref[:, 0:CO] = jnp.transpose(yv[2], (1, 0))
    tb_ref[:, CO:2 * CO] = jnp.zeros((N2, CO), jnp.float32)


def _vn2(xt, wf, wd, g, b):
    return pl.pallas_call(
        _vn2_body,
        out_shape=[
            jax.ShapeDtypeStruct((N2, 2 * CO), jnp.float32),
            jax.ShapeDtypeStruct((N2, 2 * CO), jnp.float32),
        ],
    )(xt, wf, wd, g, b)


_BQ = 256


def _knn_body(p2t_ref, p1t_ref, i0_ref, i1_ref, i2_ref,
              w0_ref, w1_ref, w2_ref):
    p1 = p1t_ref[...]
    dvs = []
    for v in range(3):
        p2c = jnp.transpose(p2t_ref[v:v + 1, :], (1, 0))
        dv = p2c - p1[v:v + 1, :]
        dvs.append(dv * dv)
    d2 = (dvs[0] + dvs[1]) + dvs[2]
    iota = lax.broadcasted_iota(jnp.int32, (N2, _BQ), 0)
    inf = jnp.float32(jnp.inf)
    irefs = [i0_ref, i1_ref, i2_ref]
    wrefs = [w0_ref, w1_ref, w2_ref]
    recips = []
    for k in range(3):
        mk = jnp.min(d2, axis=0, keepdims=True)
        idxk = jnp.min(jnp.where(d2 == mk, iota, N2), axis=0, keepdims=True)
        irefs[k][...] = idxk[0]
        if k < 2:
            d2 = jnp.where(iota == idxk, inf, d2)
        recips.append(1.0 / (jnp.sqrt(jnp.maximum(mk, 0.0)) + 1e-8))
    rs = recips[0] + recips[1] + recips[2]
    for k in range(3):
        wrefs[k][...] = (recips[k] / rs)[0]


def _knn(p2t, p1t):
    return pl.pallas_call(
        _knn_body,
        grid=(N1 // _BQ,),
        in_specs=[
            pl.BlockSpec((3, N2), lambda i: (0, 0)),
            pl.BlockSpec((3, _BQ), lambda i: (0, i)),
        ],
        out_specs=[pl.BlockSpec((_BQ,), lambda i: (i,))] * 6,
        out_shape=[jax.ShapeDtypeStruct((N1,), jnp.int32)] * 3
        + [jax.ShapeDtypeStruct((N1,), jnp.float32)] * 3,
    )(p2t, p1t)


def _interp_body(ta_hbm, tb_hbm, i0_hbm, i1_hbm, i2_hbm,
                 w0_hbm, w1_hbm, w2_hbm, oa_hbm, ob_hbm,
                 i0a, i1a, i2a, i0b, i1b, i2b,
                 ra0a, ra1a, ra2a, ra0b, ra1b, ra2b,
                 rb0a, rb1a, rb2a, rb0b, rb1b, rb2b,
                 w_v, oa_v, ob_v, sema, semb):
    wid = lax.axis_index("s") * _NC + lax.axis_index("c")
    ihbm = [i0_hbm, i1_hbm, i2_hbm]
    whbm = [w0_hbm, w1_hbm, w2_hbm]
    ibufs = [(i0a, i1a, i2a), (i0b, i1b, i2b)]
    rabufs = [(ra0a, ra1a, ra2a), (ra0b, ra1b, ra2b)]
    rbbufs = [(rb0a, rb1a, rb2a), (rb0b, rb1b, rb2b)]
    sems = [sema, semb]

    def stage(c):
        b = c % 2
        qbase = wid * _QPW + c * _CHUNK
        iv = ibufs[b]
        for k in range(3):
            pltpu.sync_copy(ihbm[k].at[pl.ds(qbase, _CHUNK)], iv[k])
            pltpu.sync_copy(whbm[k].at[pl.ds(qbase, _CHUNK)],
                            w_v.at[b, k, pl.ds(0, _CHUNK)])
        cps = [pltpu.async_copy(ta_hbm.at[iv[k]], rabufs[b][k], sems[b])
               for k in range(3)]
        cps += [pltpu.async_copy(tb_hbm.at[iv[k]], rbbufs[b][k], sems[b])
                for k in range(3)]
        return cps

    cps = stage(0)
    for c in range(_NCHUNK):
        b = c % 2
        nxt = stage(c + 1) if c + 1 < _NCHUNK else None
        for cp in cps:
            cp.wait()
        cps = nxt
        ra, rb = rabufs[b], rbbufs[b]
        qbase = wid * _QPW + c * _CHUNK

        def body(i, carry):
            w0 = w_v[b, 0, pl.ds(i, _L)][0]
            w1 = w_v[b, 1, pl.ds(i, _L)][0]
            w2 = w_v[b, 2, pl.ds(i, _L)][0]
            for f in range(2 * CO // _L):
                sl = pl.ds(f * _L, _L)
                oa_v[i, sl] = (w0 * ra[0][i, sl] + w1 * ra[1][i, sl]
                               + w2 * ra[2][i, sl])
            for f in range(CO // _L):
                sl = pl.ds(f * _L, _L)
                ob_v[i, sl] = (w0 * rb[0][i, sl] + w1 * rb[1][i, sl]
                               + w2 * rb[2][i, sl])
            return carry

        lax.fori_loop(0, _CHUNK, body, 0)
        pltpu.sync_copy(oa_v, oa_hbm.at[pl.ds(qbase, _CHUNK)])
        pltpu.sync_copy(ob_v, ob_hbm.at[pl.ds(qbase, _CHUNK)])


def _interp(ta, tb, idxs, ws):
    mesh = plsc.VectorSubcoreMesh(core_axis_name="c", subcore_axis_name="s")
    ivmem = [pltpu.VMEM((_CHUNK,), jnp.int32) for _ in range(6)]
    rvmem = [pltpu.VMEM((_CHUNK, 2 * CO), jnp.float32) for _ in range(12)]
    return pl.kernel(
        _interp_body,
        out_type=[
            jax.ShapeDtypeStruct((N1, 2 * CO), jnp.float32),
            jax.ShapeDtypeStruct((N1, 2 * CO), jnp.float32),
        ],
        mesh=mesh,
        scratch_types=ivmem + rvmem + [
            pltpu.VMEM((2, 3, _CHUNK + _L), jnp.float32),
            pltpu.VMEM((_CHUNK, 2 * CO), jnp.float32),
            pltpu.VMEM((_CHUNK, 2 * CO), jnp.float32),
            pltpu.SemaphoreType.DMA,
            pltpu.SemaphoreType.DMA,
        ],
    )(ta, tb, *idxs, *ws)


def _add_body(oa_ref, ob_ref, y1_ref, o_ref):
    o_ref[0] = y1_ref[0] + jnp.transpose(oa_ref[:, 0:CO], (1, 0))
    o_ref[1] = y1_ref[1] + jnp.transpose(oa_ref[:, CO:2 * CO], (1, 0))
    o_ref[2] = y1_ref[2] + jnp.transpose(ob_ref[:, 0:CO], (1, 0))


def _add(oa, ob, y1):
    return pl.pallas_call(
        _add_body,
        out_shape=jax.ShapeDtypeStruct((3, CO, N1), jnp.float32),
    )(oa, ob, y1)


def kernel(p1, x1, o1, p2, x2, o2, W1_feat, W1_dir, bn1_gamma, bn1_beta,
           W2_feat, W2_dir, bn2_gamma, bn2_beta):
    x1t = jnp.transpose(x1.reshape(CO, 3, N1), (1, 0, 2))
    x2t = jnp.transpose(x2.reshape(128, 3, N2), (1, 0, 2))
    ta, tb = _vn2(x2t, W2_feat, W2_dir, bn2_gamma, bn2_beta)
    y1 = _vn1(x1t, W1_feat, W1_dir, bn1_gamma, bn1_beta)
    p1t = jnp.transpose(p1, (1, 0))
    p2t = jnp.transpose(p2, (1, 0))
    p1t = lax.optimization_barrier((p1t, ta, tb))[0]
    i0, i1, i2, w0, w1, w2 = _knn(p2t, p1t)
    oa, ob = _interp(ta, tb, (i0, i1, i2), (w0, w1, w2))
    out = _add(oa, ob, y1)
    return jnp.transpose(out, (1, 0, 2)).reshape(1, CO, 3, N1)


# speedup vs baseline: 1.1271x; 1.1271x over previous
import jax
import jax.numpy as jnp
from jax import lax
from jax.experimental import pallas as pl
from jax.experimental.pallas import tpu as pltpu
from jax.experimental.pallas import tpu_sc as plsc

EPS = 1e-6

N1, N2 = 8192, 2048
CO = 64
FD = 3 * CO

_NC, _NS, _L = 2, 16, 16
_NW = _NC * _NS
_QPW = N1 // _NW
_CHUNK = 64
_NCHUNK = _QPW // _CHUNK


def _vn_math(x_ref, wf_ref, wd_ref, g_ref, b_ref):
    xv = [x_ref[v] for v in range(3)]
    pv = [jnp.dot(wf_ref[...], x, preferred_element_type=jnp.float32)
          for x in xv]
    dv = [jnp.dot(wd_ref[...], x, preferred_element_type=jnp.float32)
          for x in xv]
    nrm = jnp.sqrt(pv[0] * pv[0] + pv[1] * pv[1] + pv[2] * pv[2]) + EPS
    mean = jnp.mean(nrm, axis=1, keepdims=True)
    cen = nrm - mean
    var = jnp.mean(cen * cen, axis=1, keepdims=True)
    g = jnp.reshape(g_ref[...], (CO, 1))
    b = jnp.reshape(b_ref[...], (CO, 1))
    nbn = cen / jnp.sqrt(var + 1e-5) * g + b
    scale = nbn / nrm
    dotp = (pv[0] * dv[0] + pv[1] * dv[1] + pv[2] * dv[2]) * scale
    dnsq = dv[0] * dv[0] + dv[1] * dv[1] + dv[2] * dv[2]
    coef = jnp.where(dotp < 0, 0.8 * dotp / (dnsq + EPS), 0.0)
    return [pv[v] * scale - coef * dv[v] for v in range(3)]


def _vn1_body(x_ref, wf_ref, wd_ref, g_ref, b_ref, y_ref):
    yv = _vn_math(x_ref, wf_ref, wd_ref, g_ref, b_ref)
    for v in range(3):
        y_ref[v] = yv[v]


def _vn1(xt, wf, wd, g, b):
    return pl.pallas_call(
        _vn1_body,
        out_shape=jax.ShapeDtypeStruct((3, CO, N1), jnp.float32),
    )(xt, wf, wd, g, b)


def _vn2_body(x_ref, wf_ref, wd_ref, g_ref, b_ref, ta_ref, tb_ref):
    yv = _vn_math(x_ref, wf_ref, wd_ref, g_ref, b_ref)
    ta_ref[:, 0:CO] = jnp.transpose(yv[0], (1, 0))
    ta_ref[:, CO:2 * CO] = jnp.transpose(yv[1], (1, 0))
    tb_ref[:, 0:CO] = jnp.transpose(yv[2], (1, 0))
    tb_ref[:, CO:2 * CO] = jnp.zeros((N2, CO), jnp.float32)


def _vn2(xt, wf, wd, g, b):
    return pl.pallas_call(
        _vn2_body,
        out_shape=[
            jax.ShapeDtypeStruct((N2, 2 * CO), jnp.float32),
            jax.ShapeDtypeStruct((N2, 2 * CO), jnp.float32),
        ],
    )(xt, wf, wd, g, b)


_BQ = 256


def _knn_body(p2_ref, p1t_ref, i0_ref, i1_ref, i2_ref,
              w0_ref, w1_ref, w2_ref):
    p2 = p2_ref[...]
    p1 = p1t_ref[...]
    dvs = []
    for v in range(3):
        dv = p2[:, v:v + 1] - p1[v:v + 1, :]
        dvs.append(dv * dv)
    d2 = (dvs[0] + dvs[1]) + dvs[2]
    iota = lax.broadcasted_iota(jnp.int32, (N2, _BQ), 0)
    inf = jnp.float32(jnp.inf)
    irefs = [i0_ref, i1_ref, i2_ref]
    wrefs = [w0_ref, w1_ref, w2_ref]
    recips = []
    for k in range(3):
        mk = jnp.min(d2, axis=0, keepdims=True)
        idxk = jnp.min(jnp.where(d2 == mk, iota, N2), axis=0, keepdims=True)
        irefs[k][...] = idxk[0]
        if k < 2:
            d2 = jnp.where(iota == idxk, inf, d2)
        recips.append(1.0 / (jnp.sqrt(jnp.maximum(mk, 0.0)) + 1e-8))
    rs = recips[0] + recips[1] + recips[2]
    for k in range(3):
        wrefs[k][...] = (recips[k] / rs)[0]


def _knn(p2, p1t):
    return pl.pallas_call(
        _knn_body,
        grid=(N1 // _BQ,),
        in_specs=[
            pl.BlockSpec((N2, 3), lambda i: (0, 0)),
            pl.BlockSpec((3, _BQ), lambda i: (0, i)),
        ],
        out_specs=[pl.BlockSpec((_BQ,), lambda i: (i,))] * 6,
        out_shape=[jax.ShapeDtypeStruct((N1,), jnp.int32)] * 3
        + [jax.ShapeDtypeStruct((N1,), jnp.float32)] * 3,
    )(p2, p1t)


def _interp_body(ta_hbm, tb_hbm, i0_hbm, i1_hbm, i2_hbm,
                 w0_hbm, w1_hbm, w2_hbm, oa_hbm, ob_hbm,
                 i0a, i1a, i2a, i0b, i1b, i2b,
                 ra0a, ra1a, ra2a, ra0b, ra1b, ra2b,
                 rb0a, rb1a, rb2a, rb0b, rb1b, rb2b,
                 w_v, oa_v, ob_v, sema, semb):
    wid = lax.axis_index("s") * _NC + lax.axis_index("c")
    ihbm = [i0_hbm, i1_hbm, i2_hbm]
    whbm = [w0_hbm, w1_hbm, w2_hbm]
    ibufs = [(i0a, i1a, i2a), (i0b, i1b, i2b)]
    rabufs = [(ra0a, ra1a, ra2a), (ra0b, ra1b, ra2b)]
    rbbufs = [(rb0a, rb1a, rb2a), (rb0b, rb1b, rb2b)]
    sems = [sema, semb]

    def stage(c):
        b = c % 2
        qbase = wid * _QPW + c * _CHUNK
        iv = ibufs[b]
        for k in range(3):
            pltpu.sync_copy(ihbm[k].at[pl.ds(qbase, _CHUNK)], iv[k])
            pltpu.sync_copy(whbm[k].at[pl.ds(qbase, _CHUNK)],
                            w_v.at[b, k, pl.ds(0, _CHUNK)])
        cps = [pltpu.async_copy(ta_hbm.at[iv[k]], rabufs[b][k], sems[b])
               for k in range(3)]
        cps += [pltpu.async_copy(tb_hbm.at[iv[k]], rbbufs[b][k], sems[b])
                for k in range(3)]
        return cps

    cps = stage(0)
    for c in range(_NCHUNK):
        b = c % 2
        nxt = stage(c + 1) if c + 1 < _NCHUNK else None
        for cp in cps:
            cp.wait()
        cps = nxt
        ra, rb = rabufs[b], rbbufs[b]
        qbase = wid * _QPW + c * _CHUNK

        def body(i, carry):
            w0 = w_v[b, 0, pl.ds(i, _L)][0]
            w1 = w_v[b, 1, pl.ds(i, _L)][0]
            w2 = w_v[b, 2, pl.ds(i, _L)][0]
            for f in range(2 * CO // _L):
                sl = pl.ds(f * _L, _L)
                oa_v[i, sl] = (w0 * ra[0][i, sl] + w1 * ra[1][i, sl]
                               + w2 * ra[2][i, sl])
            for f in range(CO // _L):
                sl = pl.ds(f * _L, _L)
                ob_v[i, sl] = (w0 * rb[0][i, sl] + w1 * rb[1][i, sl]
                               + w2 * rb[2][i, sl])
            return carry

        lax.fori_loop(0, _CHUNK, body, 0)
        pltpu.sync_copy(oa_v, oa_hbm.at[pl.ds(qbase, _CHUNK)])
        pltpu.sync_copy(ob_v, ob_hbm.at[pl.ds(qbase, _CHUNK)])


def _interp(ta, tb, idxs, ws):
    mesh = plsc.VectorSubcoreMesh(core_axis_name="c", subcore_axis_name="s")
    ivmem = [pltpu.VMEM((_CHUNK,), jnp.int32) for _ in range(6)]
    rvmem = [pltpu.VMEM((_CHUNK, 2 * CO), jnp.float32) for _ in range(12)]
    return pl.kernel(
        _interp_body,
        out_type=[
            jax.ShapeDtypeStruct((N1, 2 * CO), jnp.float32),
            jax.ShapeDtypeStruct((N1, 2 * CO), jnp.float32),
        ],
        mesh=mesh,
        scratch_types=ivmem + rvmem + [
            pltpu.VMEM((2, 3, _CHUNK + _L), jnp.float32),
            pltpu.VMEM((_CHUNK, 2 * CO), jnp.float32),
            pltpu.VMEM((_CHUNK, 2 * CO), jnp.float32),
            pltpu.SemaphoreType.DMA,
            pltpu.SemaphoreType.DMA,
        ],
    )(ta, tb, *idxs, *ws)


def _add_body(oa_ref, ob_ref, y1_ref, o_ref):
    o_ref[0] = y1_ref[0] + jnp.transpose(oa_ref[:, 0:CO], (1, 0))
    o_ref[1] = y1_ref[1] + jnp.transpose(oa_ref[:, CO:2 * CO], (1, 0))
    o_ref[2] = y1_ref[2] + jnp.transpose(ob_ref[:, 0:CO], (1, 0))


def _add(oa, ob, y1):
    return pl.pallas_call(
        _add_body,
        out_shape=jax.ShapeDtypeStruct((3, CO, N1), jnp.float32),
    )(oa, ob, y1)


def kernel(p1, x1, o1, p2, x2, o2, W1_feat, W1_dir, bn1_gamma, bn1_beta,
           W2_feat, W2_dir, bn2_gamma, bn2_beta):
    x1t = jnp.transpose(x1.reshape(CO, 3, N1), (1, 0, 2))
    x2t = jnp.transpose(x2.reshape(128, 3, N2), (1, 0, 2))
    ta, tb = _vn2(x2t, W2_feat, W2_dir, bn2_gamma, bn2_beta)
    y1 = _vn1(x1t, W1_feat, W1_dir, bn1_gamma, bn1_beta)
    p1t = jnp.transpose(p1, (1, 0))
    p1t = lax.optimization_barrier((p1t, ta, tb))[0]
    i0, i1, i2, w0, w1, w2 = _knn(p2, p1t)
    oa, ob = _interp(ta, tb, (i0, i1, i2), (w0, w1, w2))
    out = _add(oa, ob, y1)
    return jnp.transpose(out, (1, 0, 2)).reshape(1, CO, 3, N1)


# speedup vs baseline: 1.2875x; 1.1423x over previous
import jax
import jax.numpy as jnp
from jax import lax
from jax.experimental import pallas as pl
from jax.experimental.pallas import tpu as pltpu
from jax.experimental.pallas import tpu_sc as plsc

EPS = 1e-6

N1, N2 = 8192, 2048
CO = 64
FD = 3 * CO

_NC, _NS, _L = 2, 16, 16
_NW = _NC * _NS
_QPW = N1 // _NW
_CHUNK = 64
_NCHUNK = _QPW // _CHUNK


def _vn_math(x_ref, wf_ref, wd_ref, g_ref, b_ref):
    xv = [x_ref[v] for v in range(3)]
    pv = [jnp.dot(wf_ref[...], x, preferred_element_type=jnp.float32)
          for x in xv]
    dv = [jnp.dot(wd_ref[...], x, preferred_element_type=jnp.float32)
          for x in xv]
    nrm = jnp.sqrt(pv[0] * pv[0] + pv[1] * pv[1] + pv[2] * pv[2]) + EPS
    mean = jnp.mean(nrm, axis=1, keepdims=True)
    cen = nrm - mean
    var = jnp.mean(cen * cen, axis=1, keepdims=True)
    g = jnp.reshape(g_ref[...], (CO, 1))
    b = jnp.reshape(b_ref[...], (CO, 1))
    nbn = cen / jnp.sqrt(var + 1e-5) * g + b
    scale = nbn / nrm
    dotp = (pv[0] * dv[0] + pv[1] * dv[1] + pv[2] * dv[2]) * scale
    dnsq = dv[0] * dv[0] + dv[1] * dv[1] + dv[2] * dv[2]
    coef = jnp.where(dotp < 0, 0.8 * dotp / (dnsq + EPS), 0.0)
    return [pv[v] * scale - coef * dv[v] for v in range(3)]


def _vn1_body(x_ref, wf_ref, wd_ref, g_ref, b_ref, y_ref):
    yv = _vn_math(x_ref, wf_ref, wd_ref, g_ref, b_ref)
    for v in range(3):
        y_ref[v] = yv[v]


def _vn1(xt, wf, wd, g, b):
    return pl.pallas_call(
        _vn1_body,
        out_shape=jax.ShapeDtypeStruct((3, CO, N1), jnp.float32),
    )(xt, wf, wd, g, b)


def _vn2_body(x_ref, wf_ref, wd_ref, g_ref, b_ref, ta_ref, tb_ref):
    yv = _vn_math(x_ref, wf_ref, wd_ref, g_ref, b_ref)
    ta_ref[:, 0:CO] = jnp.transpose(yv[0], (1, 0))
    ta_ref[:, CO:2 * CO] = jnp.transpose(yv[1], (1, 0))
    tb_ref[:, 0:CO] = jnp.transpose(yv[2], (1, 0))
    tb_ref[:, CO:2 * CO] = jnp.zeros((N2, CO), jnp.float32)


def _vn2(xt, wf, wd, g, b):
    return pl.pallas_call(
        _vn2_body,
        out_shape=[
            jax.ShapeDtypeStruct((N2, 2 * CO), jnp.float32),
            jax.ShapeDtypeStruct((N2, 2 * CO), jnp.float32),
        ],
    )(xt, wf, wd, g, b)


_BQ = 512


def _knn_body(p2_ref, p1t_ref, i0_ref, i1_ref, i2_ref,
              w0_ref, w1_ref, w2_ref):
    p2 = p2_ref[...]
    p1 = p1t_ref[...]
    dvs = []
    for v in range(3):
        dv = p2[:, v:v + 1] - p1[v:v + 1, :]
        dvs.append(dv * dv)
    d2 = (dvs[0] + dvs[1]) + dvs[2]
    iota = lax.broadcasted_iota(jnp.int32, (N2, _BQ), 0)
    inf = jnp.float32(jnp.inf)
    irefs = [i0_ref, i1_ref, i2_ref]
    wrefs = [w0_ref, w1_ref, w2_ref]
    recips = []
    for k in range(3):
        mk = jnp.min(d2, axis=0, keepdims=True)
        idxk = jnp.min(jnp.where(d2 == mk, iota, N2), axis=0, keepdims=True)
        irefs[k][...] = idxk[0]
        if k < 2:
            d2 = jnp.where(iota == idxk, inf, d2)
        recips.append(1.0 / (jnp.sqrt(jnp.maximum(mk, 0.0)) + 1e-8))
    rs = recips[0] + recips[1] + recips[2]
    for k in range(3):
        wrefs[k][...] = (recips[k] / rs)[0]


def _knn(p2, p1t):
    return pl.pallas_call(
        _knn_body,
        grid=(N1 // _BQ,),
        in_specs=[
            pl.BlockSpec((N2, 3), lambda i: (0, 0)),
            pl.BlockSpec((3, _BQ), lambda i: (0, i)),
        ],
        out_specs=[pl.BlockSpec((_BQ,), lambda i: (i,))] * 6,
        out_shape=[jax.ShapeDtypeStruct((N1,), jnp.int32)] * 3
        + [jax.ShapeDtypeStruct((N1,), jnp.float32)] * 3,
    )(p2, p1t)


def _interp_body(ta_hbm, tb_hbm, i0_hbm, i1_hbm, i2_hbm,
                 w0_hbm, w1_hbm, w2_hbm, oa_hbm, ob_hbm,
                 i0a, i1a, i2a, i0b, i1b, i2b,
                 ra0a, ra1a, ra2a, ra0b, ra1b, ra2b,
                 rb0a, rb1a, rb2a, rb0b, rb1b, rb2b,
                 w_v, oa_v, ob_v, sema, semb):
    wid = lax.axis_index("s") * _NC + lax.axis_index("c")
    ihbm = [i0_hbm, i1_hbm, i2_hbm]
    whbm = [w0_hbm, w1_hbm, w2_hbm]
    ibufs = [(i0a, i1a, i2a), (i0b, i1b, i2b)]
    rabufs = [(ra0a, ra1a, ra2a), (ra0b, ra1b, ra2b)]
    rbbufs = [(rb0a, rb1a, rb2a), (rb0b, rb1b, rb2b)]
    sems = [sema, semb]

    def stage(c):
        b = c % 2
        qbase = wid * _QPW + c * _CHUNK
        iv = ibufs[b]
        for k in range(3):
            pltpu.sync_copy(ihbm[k].at[pl.ds(qbase, _CHUNK)], iv[k])
            pltpu.sync_copy(whbm[k].at[pl.ds(qbase, _CHUNK)],
                            w_v.at[b, k, pl.ds(0, _CHUNK)])
        cps = [pltpu.async_copy(ta_hbm.at[iv[k]], rabufs[b][k], sems[b])
               for k in range(3)]
        cps += [pltpu.async_copy(tb_hbm.at[iv[k]], rbbufs[b][k], sems[b])
                for k in range(3)]
        return cps

    cps = stage(0)
    for c in range(_NCHUNK):
        b = c % 2
        nxt = stage(c + 1) if c + 1 < _NCHUNK else None
        for cp in cps:
            cp.wait()
        cps = nxt
        ra, rb = rabufs[b], rbbufs[b]
        qbase = wid * _QPW + c * _CHUNK

        def body(iq, carry):
            for u in range(4):
                i = iq * 4 + u
                w0 = w_v[b, 0, pl.ds(i, _L)][0]
                w1 = w_v[b, 1, pl.ds(i, _L)][0]
                w2 = w_v[b, 2, pl.ds(i, _L)][0]
                for f in range(2 * CO // _L):
                    sl = pl.ds(f * _L, _L)
                    oa_v[i, sl] = (w0 * ra[0][i, sl] + w1 * ra[1][i, sl]
                                   + w2 * ra[2][i, sl])
                for f in range(CO // _L):
                    sl = pl.ds(f * _L, _L)
                    ob_v[i, sl] = (w0 * rb[0][i, sl] + w1 * rb[1][i, sl]
                                   + w2 * rb[2][i, sl])
            return carry

        lax.fori_loop(0, _CHUNK // 4, body, 0)
        pltpu.sync_copy(oa_v, oa_hbm.at[pl.ds(qbase, _CHUNK)])
        pltpu.sync_copy(ob_v, ob_hbm.at[pl.ds(qbase, _CHUNK)])


def _interp(ta, tb, idxs, ws):
    mesh = plsc.VectorSubcoreMesh(core_axis_name="c", subcore_axis_name="s")
    ivmem = [pltpu.VMEM((_CHUNK,), jnp.int32) for _ in range(6)]
    rvmem = [pltpu.VMEM((_CHUNK, 2 * CO), jnp.float32) for _ in range(12)]
    return pl.kernel(
        _interp_body,
        out_type=[
            jax.ShapeDtypeStruct((N1, 2 * CO), jnp.float32),
            jax.ShapeDtypeStruct((N1, 2 * CO), jnp.float32),
        ],
        mesh=mesh,
        scratch_types=ivmem + rvmem + [
            pltpu.VMEM((2, 3, _CHUNK + _L), jnp.float32),
            pltpu.VMEM((_CHUNK, 2 * CO), jnp.float32),
            pltpu.VMEM((_CHUNK, 2 * CO), jnp.float32),
            pltpu.SemaphoreType.DMA,
            pltpu.SemaphoreType.DMA,
        ],
    )(ta, tb, *idxs, *ws)


def _add_body(oa_ref, ob_ref, y1_ref, o_ref):
    o_ref[0] = y1_ref[0] + jnp.transpose(oa_ref[:, 0:CO], (1, 0))
    o_ref[1] = y1_ref[1] + jnp.transpose(oa_ref[:, CO:2 * CO], (1, 0))
    o_ref[2] = y1_ref[2] + jnp.transpose(ob_ref[:, 0:CO], (1, 0))


def _add(oa, ob, y1):
    return pl.pallas_call(
        _add_body,
        out_shape=jax.ShapeDtypeStruct((3, CO, N1), jnp.float32),
    )(oa, ob, y1)


def kernel(p1, x1, o1, p2, x2, o2, W1_feat, W1_dir, bn1_gamma, bn1_beta,
           W2_feat, W2_dir, bn2_gamma, bn2_beta):
    x1t = jnp.transpose(x1.reshape(CO, 3, N1), (1, 0, 2))
    x2t = jnp.transpose(x2.reshape(128, 3, N2), (1, 0, 2))
    ta, tb = _vn2(x2t, W2_feat, W2_dir, bn2_gamma, bn2_beta)
    y1 = _vn1(x1t, W1_feat, W1_dir, bn1_gamma, bn1_beta)
    p1t = jnp.transpose(p1, (1, 0))
    p1t = lax.optimization_barrier((p1t, ta, tb))[0]
    i0, i1, i2, w0, w1, w2 = _knn(p2, p1t)
    oa, ob = _interp(ta, tb, (i0, i1, i2), (w0, w1, w2))
    out = _add(oa, ob, y1)
    return jnp.transpose(out, (1, 0, 2)).reshape(1, CO, 3, N1)


# speedup vs baseline: 1.2947x; 1.0056x over previous
import jax
import jax.numpy as jnp
from jax import lax
from jax.experimental import pallas as pl
from jax.experimental.pallas import tpu as pltpu
from jax.experimental.pallas import tpu_sc as plsc

EPS = 1e-6

N1, N2 = 8192, 2048
CO = 64
FD = 3 * CO

_NC, _NS, _L = 2, 16, 16
_NW = _NC * _NS
_QPW = N1 // _NW
_CHUNK = 64
_NCHUNK = _QPW // _CHUNK


def _vn_math(x_ref, wf_ref, wd_ref, g_ref, b_ref):
    xv = [x_ref[v] for v in range(3)]
    pv = [jnp.dot(wf_ref[...], x, preferred_element_type=jnp.float32)
          for x in xv]
    dv = [jnp.dot(wd_ref[...], x, preferred_element_type=jnp.float32)
          for x in xv]
    nrm = jnp.sqrt(pv[0] * pv[0] + pv[1] * pv[1] + pv[2] * pv[2]) + EPS
    mean = jnp.mean(nrm, axis=1, keepdims=True)
    cen = nrm - mean
    var = jnp.mean(cen * cen, axis=1, keepdims=True)
    g = jnp.reshape(g_ref[...], (CO, 1))
    b = jnp.reshape(b_ref[...], (CO, 1))
    nbn = cen / jnp.sqrt(var + 1e-5) * g + b
    scale = nbn / nrm
    dotp = (pv[0] * dv[0] + pv[1] * dv[1] + pv[2] * dv[2]) * scale
    dnsq = dv[0] * dv[0] + dv[1] * dv[1] + dv[2] * dv[2]
    coef = jnp.where(dotp < 0, 0.8 * dotp / (dnsq + EPS), 0.0)
    return [pv[v] * scale - coef * dv[v] for v in range(3)]


def _vn1_body(x_ref, wf_ref, wd_ref, g_ref, b_ref, y_ref):
    yv = _vn_math(x_ref, wf_ref, wd_ref, g_ref, b_ref)
    for v in range(3):
        y_ref[v] = yv[v]


def _vn1(xt, wf, wd, g, b):
    return pl.pallas_call(
        _vn1_body,
        out_shape=jax.ShapeDtypeStruct((3, CO, N1), jnp.float32),
    )(xt, wf, wd, g, b)


def _vn2_body(x_ref, wf_ref, wd_ref, g_ref, b_ref, ta_ref, tb_ref):
    yv = _vn_math(x_ref, wf_ref, wd_ref, g_ref, b_ref)
    ta_ref[:, 0:CO] = jnp.transpose(yv[0], (1, 0))
    ta_ref[:, CO:2 * CO] = jnp.transpose(yv[1], (1, 0))
    tb_ref[:, 0:CO] = jnp.transpose(yv[2], (1, 0))
    tb_ref[:, CO:2 * CO] = jnp.zeros((N2, CO), jnp.float32)


def _vn2(xt, wf, wd, g, b):
    return pl.pallas_call(
        _vn2_body,
        out_shape=[
            jax.ShapeDtypeStruct((N2, 2 * CO), jnp.float32),
            jax.ShapeDtypeStruct((N2, 2 * CO), jnp.float32),
        ],
    )(xt, wf, wd, g, b)


_BQ = 1024


def _knn_body(p2_ref, p1t_ref, i0_ref, i1_ref, i2_ref,
              w0_ref, w1_ref, w2_ref):
    p2 = p2_ref[...]
    p1 = p1t_ref[...]
    dvs = []
    for v in range(3):
        dv = p2[:, v:v + 1] - p1[v:v + 1, :]
        dvs.append(dv * dv)
    d2 = (dvs[0] + dvs[1]) + dvs[2]
    iota = lax.broadcasted_iota(jnp.int32, (N2, _BQ), 0)
    inf = jnp.float32(jnp.inf)
    irefs = [i0_ref, i1_ref, i2_ref]
    wrefs = [w0_ref, w1_ref, w2_ref]
    recips = []
    for k in range(3):
        mk = jnp.min(d2, axis=0, keepdims=True)
        idxk = jnp.min(jnp.where(d2 == mk, iota, N2), axis=0, keepdims=True)
        irefs[k][...] = idxk[0]
        if k < 2:
            d2 = jnp.where(iota == idxk, inf, d2)
        recips.append(1.0 / (jnp.sqrt(jnp.maximum(mk, 0.0)) + 1e-8))
    rs = recips[0] + recips[1] + recips[2]
    for k in range(3):
        wrefs[k][...] = (recips[k] / rs)[0]


def _knn(p2, p1t):
    return pl.pallas_call(
        _knn_body,
        grid=(N1 // _BQ,),
        in_specs=[
            pl.BlockSpec((N2, 3), lambda i: (0, 0)),
            pl.BlockSpec((3, _BQ), lambda i: (0, i)),
        ],
        out_specs=[pl.BlockSpec((_BQ,), lambda i: (i,))] * 6,
        out_shape=[jax.ShapeDtypeStruct((N1,), jnp.int32)] * 3
        + [jax.ShapeDtypeStruct((N1,), jnp.float32)] * 3,
    )(p2, p1t)


def _interp_body(ta_hbm, tb_hbm, i0_hbm, i1_hbm, i2_hbm,
                 w0_hbm, w1_hbm, w2_hbm, oa_hbm, ob_hbm,
                 i0a, i1a, i2a, i0b, i1b, i2b,
                 ra0a, ra1a, ra2a, ra0b, ra1b, ra2b,
                 rb0a, rb1a, rb2a, rb0b, rb1b, rb2b,
                 w_v, oa_v, ob_v, sema, semb):
    wid = lax.axis_index("s") * _NC + lax.axis_index("c")
    ihbm = [i0_hbm, i1_hbm, i2_hbm]
    whbm = [w0_hbm, w1_hbm, w2_hbm]
    ibufs = [(i0a, i1a, i2a), (i0b, i1b, i2b)]
    rabufs = [(ra0a, ra1a, ra2a), (ra0b, ra1b, ra2b)]
    rbbufs = [(rb0a, rb1a, rb2a), (rb0b, rb1b, rb2b)]
    sems = [sema, semb]

    def stage(c):
        b = c % 2
        qbase = wid * _QPW + c * _CHUNK
        iv = ibufs[b]
        for k in range(3):
            pltpu.sync_copy(ihbm[k].at[pl.ds(qbase, _CHUNK)], iv[k])
            pltpu.sync_copy(whbm[k].at[pl.ds(qbase, _CHUNK)],
                            w_v.at[b, k, pl.ds(0, _CHUNK)])
        cps = [pltpu.async_copy(ta_hbm.at[iv[k]], rabufs[b][k], sems[b])
               for k in range(3)]
        cps += [pltpu.async_copy(tb_hbm.at[iv[k]], rbbufs[b][k], sems[b])
                for k in range(3)]
        return cps

    cps = stage(0)
    for c in range(_NCHUNK):
        b = c % 2
        nxt = stage(c + 1) if c + 1 < _NCHUNK else None
        for cp in cps:
            cp.wait()
        cps = nxt
        ra, rb = rabufs[b], rbbufs[b]
        qbase = wid * _QPW + c * _CHUNK

        def body(iq, carry):
            for u in range(4):
                i = iq * 4 + u
                w0 = w_v[b, 0, pl.ds(i, _L)][0]
                w1 = w_v[b, 1, pl.ds(i, _L)][0]
                w2 = w_v[b, 2, pl.ds(i, _L)][0]
                for f in range(2 * CO // _L):
                    sl = pl.ds(f * _L, _L)
                    oa_v[i, sl] = (w0 * ra[0][i, sl] + w1 * ra[1][i, sl]
                                   + w2 * ra[2][i, sl])
                for f in range(CO // _L):
                    sl = pl.ds(f * _L, _L)
                    ob_v[i, sl] = (w0 * rb[0][i, sl] + w1 * rb[1][i, sl]
                                   + w2 * rb[2][i, sl])
            return carry

        lax.fori_loop(0, _CHUNK // 4, body, 0)
        pltpu.sync_copy(oa_v, oa_hbm.at[pl.ds(qbase, _CHUNK)])
        pltpu.sync_copy(ob_v, ob_hbm.at[pl.ds(qbase, _CHUNK)])


def _interp(ta, tb, idxs, ws):
    mesh = plsc.VectorSubcoreMesh(core_axis_name="c", subcore_axis_name="s")
    ivmem = [pltpu.VMEM((_CHUNK,), jnp.int32) for _ in range(6)]
    rvmem = [pltpu.VMEM((_CHUNK, 2 * CO), jnp.float32) for _ in range(12)]
    return pl.kernel(
        _interp_body,
        out_type=[
            jax.ShapeDtypeStruct((N1, 2 * CO), jnp.float32),
            jax.ShapeDtypeStruct((N1, 2 * CO), jnp.float32),
        ],
        mesh=mesh,
        scratch_types=ivmem + rvmem + [
            pltpu.VMEM((2, 3, _CHUNK + _L), jnp.float32),
            pltpu.VMEM((_CHUNK, 2 * CO), jnp.float32),
            pltpu.VMEM((_CHUNK, 2 * CO), jnp.float32),
            pltpu.SemaphoreType.DMA,
            pltpu.SemaphoreType.DMA,
        ],
    )(ta, tb, *idxs, *ws)


def _add_body(oa_ref, ob_ref, y1_ref, o_ref):
    o_ref[0] = y1_ref[0] + jnp.transpose(oa_ref[:, 0:CO], (1, 0))
    o_ref[1] = y1_ref[1] + jnp.transpose(oa_ref[:, CO:2 * CO], (1, 0))
    o_ref[2] = y1_ref[2] + jnp.transpose(ob_ref[:, 0:CO], (1, 0))


def _add(oa, ob, y1):
    return pl.pallas_call(
        _add_body,
        out_shape=jax.ShapeDtypeStruct((3, CO, N1), jnp.float32),
    )(oa, ob, y1)


def kernel(p1, x1, o1, p2, x2, o2, W1_feat, W1_dir, bn1_gamma, bn1_beta,
           W2_feat, W2_dir, bn2_gamma, bn2_beta):
    x1t = jnp.transpose(x1.reshape(CO, 3, N1), (1, 0, 2))
    x2t = jnp.transpose(x2.reshape(128, 3, N2), (1, 0, 2))
    ta, tb = _vn2(x2t, W2_feat, W2_dir, bn2_gamma, bn2_beta)
    y1 = _vn1(x1t, W1_feat, W1_dir, bn1_gamma, bn1_beta)
    p1t = jnp.transpose(p1, (1, 0))
    p1t = lax.optimization_barrier((p1t, ta, tb))[0]
    i0, i1, i2, w0, w1, w2 = _knn(p2, p1t)
    oa, ob = _interp(ta, tb, (i0, i1, i2), (w0, w1, w2))
    out = _add(oa, ob, y1)
    return jnp.transpose(out, (1, 0, 2)).reshape(1, CO, 3, N1)
